# baseline (speedup 1.0000x reference)
import functools

import jax
import jax.numpy as jnp
from jax import lax
from jax.experimental import pallas as pl
from jax.experimental.pallas import tpu as pltpu
from jax.experimental.pallas import tpu_sc as plsc

NUM_CORES = 2
NUM_SUBCORES = 16
NUM_WORKERS = NUM_CORES * NUM_SUBCORES
CHUNK = 40
WIN = 25
NBUF = 5


def _sc_aggregate(x, edges, n_windows, n_pad):
    n, d = x.shape
    rows_per_sub = n_pad // NUM_SUBCORES

    mesh = plsc.VectorSubcoreMesh(core_axis_name="c", subcore_axis_name="s")

    @functools.partial(
        pl.kernel,
        out_type=jax.ShapeDtypeStruct((NUM_CORES, n_pad, d), jnp.float32),
        mesh=mesh,
        scratch_types=[
            pltpu.VMEM((WIN, CHUNK), jnp.int32),
            pltpu.VMEM((WIN, CHUNK), jnp.int32),
            pltpu.VMEM((WIN, CHUNK), jnp.int32),
            pltpu.VMEM((WIN, CHUNK), jnp.int32),
            pltpu.VMEM((NBUF, CHUNK, d), jnp.float32),
            pltpu.VMEM_SHARED((n_pad, d), jnp.float32),
            pltpu.SemaphoreType.DMA,
            pltpu.SemaphoreType.DMA,
            pltpu.SemaphoreType.DMA,
            pltpu.SemaphoreType.DMA,
            pltpu.SemaphoreType.DMA,
            pltpu.SemaphoreType.DMA,
            pltpu.SemaphoreType.DMA,
            pltpu.SemaphoreType.DMA,
            pltpu.SemaphoreType.DMA,
            pltpu.SemaphoreType.DMA,
            pltpu.SemaphoreType.DMA,
            pltpu.SemaphoreType.DMA,
        ],
    )
    def sc_agg(x_hbm, e_hbm, out_hbm,
               src_w0, src_w1, dst_w0, dst_w1, rows, agg_sh,
               sem0, sem1, sem2, sem3, sem4,
               ssem0, ssem1, ssem2, ssem3, ssem4, sem_sw, sem_dw):
        cid = lax.axis_index("c")
        sid = lax.axis_index("s")
        wid = sid * NUM_CORES + cid

        swin = (src_w0, src_w1)
        dwin = (dst_w0, dst_w1)
        bufs = tuple(rows.at[i] for i in range(NBUF))
        sems = (sem0, sem1, sem2, sem3, sem4)
        ssems = (ssem0, ssem1, ssem2, ssem3, ssem4)

        def stage(w, p):
            pltpu.async_copy(e_hbm.at[0, wid, w], swin[p], sem_sw)
            pltpu.async_copy(e_hbm.at[1, wid, w], dwin[p], sem_dw)

        def stage_wait(w, p):
            pltpu.make_async_copy(e_hbm.at[0, wid, w], swin[p],
                                  sem_sw).wait()
            pltpu.make_async_copy(e_hbm.at[1, wid, w], dwin[p],
                                  sem_dw).wait()

        def gather(sbuf, j, i):
            pltpu.async_copy(x_hbm.at[sbuf.at[j]], bufs[i], sems[i])

        def gather_wait(sbuf, j, i):
            pltpu.make_async_copy(x_hbm.at[sbuf.at[j]], bufs[i],
                                  sems[i]).wait()

        def scatter_start(dbuf, j, i):
            pltpu.async_copy(bufs[i], agg_sh.at[dbuf.at[j]], ssems[i],
                             add=True)

        def scatter_wait(dbuf, j, i):
            pltpu.make_async_copy(bufs[i], agg_sh.at[dbuf.at[j]],
                                  ssems[i]).wait()

        stage(0, 0)
        stage(1, 1)

        zvec = jnp.zeros((16,), jnp.float32)

        @pl.loop(0, CHUNK)
        def _(i):
            @pl.loop(0, d, step=16)
            def _(j):
                rows[0, i, pl.ds(j, 16)] = zvec

        @pl.loop(0, rows_per_sub // CHUNK)
        def _(k):
            pltpu.sync_copy(bufs[0],
                            agg_sh.at[pl.ds(sid * rows_per_sub + k * CHUNK,
                                            CHUNK)])

        stage_wait(0, 0)
        for i in range(NBUF):
            gather(swin[0], i, i)

        plsc.subcore_barrier()

        for w in range(n_windows):
            sb, db = swin[w % 2], dwin[w % 2]

            @pl.loop(0, WIN - NBUF, step=NBUF)
            def _(j, sb=sb, db=db):
                for i in range(NBUF):
                    gather_wait(sb, j + i, i)
                    scatter_start(db, j + i, i)
                for i in range(NBUF):
                    scatter_wait(db, j + i, i)
                    gather(sb, j + NBUF + i, i)

            if w + 1 < n_windows:
                nsb = swin[(w + 1) % 2]
                stage_wait(w + 1, (w + 1) % 2)
                for i in range(NBUF):
                    gather_wait(sb, WIN - NBUF + i, i)
                    scatter_start(db, WIN - NBUF + i, i)
                for i in range(NBUF):
                    scatter_wait(db, WIN - NBUF + i, i)
                    gather(nsb, i, i)
                if w + 2 < n_windows:
                    stage(w + 2, w % 2)
            else:
                for i in range(NBUF):
                    gather_wait(sb, WIN - NBUF + i, i)
                    scatter_start(db, WIN - NBUF + i, i)
                for i in range(NBUF):
                    scatter_wait(db, WIN - NBUF + i, i)

        plsc.subcore_barrier()

        pltpu.sync_copy(agg_sh.at[pl.ds(sid * rows_per_sub, rows_per_sub)],
                        out_hbm.at[cid, pl.ds(sid * rows_per_sub,
                                              rows_per_sub)])

    return sc_agg(x, edges)


def _tc_finish(partials, W, b2d, n):
    _, _, d = partials.shape
    blk = 5000

    def body(p_ref, w_ref, b_ref, o_ref):
        agg = p_ref[0] + p_ref[1]
        y = lax.dot_general(agg, w_ref[...], (((1,), (1,)), ((), ())),
                            preferred_element_type=jnp.float32)
        o_ref[...] = jnp.maximum(y + b_ref[...], 0.0)

    return pl.pallas_call(
        body,
        grid=(n // blk,),
        in_specs=[
            pl.BlockSpec((2, blk, d), lambda i: (0, i, 0)),
            pl.BlockSpec((d, d), lambda i: (0, 0)),
            pl.BlockSpec((1, d), lambda i: (0, 0)),
        ],
        out_specs=pl.BlockSpec((blk, d), lambda i: (i, 0)),
        out_shape=jax.ShapeDtypeStruct((n, d), jnp.float32),
    )(partials, W, b2d)


def kernel(x, edge_index, W, b):
    n, d = x.shape
    e = edge_index.shape[1]
    per_worker = e // NUM_WORKERS
    n_chunks = per_worker // CHUNK
    n_windows = n_chunks // WIN
    assert per_worker * NUM_WORKERS == e
    assert n_chunks * CHUNK == per_worker and n_windows * WIN == n_chunks

    rows_per_sub = (-(-n // NUM_SUBCORES) + CHUNK - 1) // CHUNK * CHUNK
    n_pad = rows_per_sub * NUM_SUBCORES

    edges = edge_index.astype(jnp.int32).reshape(
        2, NUM_WORKERS, n_windows, WIN, CHUNK)

    partials = _sc_aggregate(x, edges, n_windows, n_pad)
    return _tc_finish(partials, W, b.reshape(1, d), n)


# speedup vs baseline: 1.1551x; 1.1551x over previous
import functools

import jax
import jax.numpy as jnp
from jax import lax
from jax.experimental import pallas as pl
from jax.experimental.pallas import tpu as pltpu
from jax.experimental.pallas import tpu_sc as plsc

NUM_CORES = 2
NUM_SUBCORES = 16
NUM_WORKERS = NUM_CORES * NUM_SUBCORES
CHUNK = 40
WIN = 25
NBUF = 5


def _sc_aggregate(x, edges, n_windows, n_pad):
    n, d = x.shape
    rows_per_sub = n_pad // NUM_SUBCORES

    mesh = plsc.VectorSubcoreMesh(core_axis_name="c", subcore_axis_name="s")

    @functools.partial(
        pl.kernel,
        out_type=jax.ShapeDtypeStruct((NUM_CORES, n_pad, d), jnp.float32),
        mesh=mesh,
        scratch_types=[
            pltpu.VMEM((WIN, CHUNK), jnp.int32),
            pltpu.VMEM((WIN, CHUNK), jnp.int32),
            pltpu.VMEM((WIN, CHUNK), jnp.int32),
            pltpu.VMEM((WIN, CHUNK), jnp.int32),
            pltpu.VMEM((NBUF, CHUNK, d), jnp.float32),
            pltpu.VMEM_SHARED((n_pad, d), jnp.float32),
            pltpu.SemaphoreType.DMA,
            pltpu.SemaphoreType.DMA,
            pltpu.SemaphoreType.DMA,
            pltpu.SemaphoreType.DMA,
            pltpu.SemaphoreType.DMA,
            pltpu.SemaphoreType.DMA,
            pltpu.SemaphoreType.DMA,
            pltpu.SemaphoreType.DMA,
            pltpu.SemaphoreType.DMA,
            pltpu.SemaphoreType.DMA,
            pltpu.SemaphoreType.DMA,
            pltpu.SemaphoreType.DMA,
        ],
    )
    def sc_agg(x_hbm, e_hbm, out_hbm,
               src_w0, src_w1, dst_w0, dst_w1, rows, agg_sh,
               sem0, sem1, sem2, sem3, sem4,
               ssem0, ssem1, ssem2, ssem3, ssem4, sem_sw, sem_dw):
        cid = lax.axis_index("c")
        sid = lax.axis_index("s")
        wid = sid * NUM_CORES + cid

        swin = (src_w0, src_w1)
        dwin = (dst_w0, dst_w1)
        bufs = tuple(rows.at[i] for i in range(NBUF))
        sems = (sem0, sem1, sem2, sem3, sem4)
        ssems = (ssem0, ssem1, ssem2, ssem3, ssem4)

        def stage(w, p):
            pltpu.async_copy(e_hbm.at[0, wid, w], swin[p], sem_sw)
            pltpu.async_copy(e_hbm.at[1, wid, w], dwin[p], sem_dw)

        def stage_wait(w, p):
            pltpu.make_async_copy(e_hbm.at[0, wid, w], swin[p],
                                  sem_sw).wait()
            pltpu.make_async_copy(e_hbm.at[1, wid, w], dwin[p],
                                  sem_dw).wait()

        def gather(sbuf, j, i):
            pltpu.async_copy(x_hbm.at[sbuf.at[j]], bufs[i], sems[i])

        def gather_wait(sbuf, j, i):
            pltpu.make_async_copy(x_hbm.at[sbuf.at[j]], bufs[i],
                                  sems[i]).wait()

        def scatter(dbuf, j, i):
            pltpu.sync_copy(bufs[i], agg_sh.at[dbuf.at[j]], add=True)

        stage(0, 0)
        stage(1, 1)

        zvec = jnp.zeros((16,), jnp.float32)

        @pl.loop(0, CHUNK)
        def _(i):
            @pl.loop(0, d, step=16)
            def _(j):
                rows[0, i, pl.ds(j, 16)] = zvec

        @pl.loop(0, rows_per_sub // CHUNK)
        def _(k):
            pltpu.sync_copy(bufs[0],
                            agg_sh.at[pl.ds(sid * rows_per_sub + k * CHUNK,
                                            CHUNK)])

        stage_wait(0, 0)
        for i in range(NBUF):
            gather(swin[0], i, i)

        plsc.subcore_barrier()

        for w in range(n_windows):
            sb, db = swin[w % 2], dwin[w % 2]

            @pl.loop(0, WIN - NBUF, step=NBUF)
            def _(j, sb=sb, db=db):
                for i in range(NBUF):
                    gather_wait(sb, j + i, i)
                    scatter(db, j + i, i)
                    gather(sb, j + NBUF + i, i)

            if w + 1 < n_windows:
                nsb = swin[(w + 1) % 2]
                stage_wait(w + 1, (w + 1) % 2)
                for i in range(NBUF):
                    gather_wait(sb, WIN - NBUF + i, i)
                    scatter(db, WIN - NBUF + i, i)
                    gather(nsb, i, i)
                if w + 2 < n_windows:
                    stage(w + 2, w % 2)
            else:
                for i in range(NBUF):
                    gather_wait(sb, WIN - NBUF + i, i)
                    scatter(db, WIN - NBUF + i, i)

        plsc.subcore_barrier()

        pltpu.sync_copy(agg_sh.at[pl.ds(sid * rows_per_sub, rows_per_sub)],
                        out_hbm.at[cid, pl.ds(sid * rows_per_sub,
                                              rows_per_sub)])

    return sc_agg(x, edges)


def _tc_finish(partials, W, b2d, n):
    _, _, d = partials.shape
    blk = 5000

    def body(p_ref, w_ref, b_ref, o_ref):
        agg = p_ref[0] + p_ref[1]
        y = lax.dot_general(agg, w_ref[...], (((1,), (1,)), ((), ())),
                            preferred_element_type=jnp.float32)
        o_ref[...] = jnp.maximum(y + b_ref[...], 0.0)

    return pl.pallas_call(
        body,
        grid=(n // blk,),
        in_specs=[
            pl.BlockSpec((2, blk, d), lambda i: (0, i, 0)),
            pl.BlockSpec((d, d), lambda i: (0, 0)),
            pl.BlockSpec((1, d), lambda i: (0, 0)),
        ],
        out_specs=pl.BlockSpec((blk, d), lambda i: (i, 0)),
        out_shape=jax.ShapeDtypeStruct((n, d), jnp.float32),
    )(partials, W, b2d)


def kernel(x, edge_index, W, b):
    n, d = x.shape
    e = edge_index.shape[1]
    per_worker = e // NUM_WORKERS
    n_chunks = per_worker // CHUNK
    n_windows = n_chunks // WIN
    assert per_worker * NUM_WORKERS == e
    assert n_chunks * CHUNK == per_worker and n_windows * WIN == n_chunks

    rows_per_sub = (-(-n // NUM_SUBCORES) + CHUNK - 1) // CHUNK * CHUNK
    n_pad = rows_per_sub * NUM_SUBCORES

    edges = edge_index.astype(jnp.int32).reshape(
        2, NUM_WORKERS, n_windows, WIN, CHUNK)

    partials = _sc_aggregate(x, edges, n_windows, n_pad)
    return _tc_finish(partials, W, b.reshape(1, d), n)
